# baseline (speedup 1.0000x reference)
import functools

import jax
import jax.numpy as jnp
from jax import lax
from jax.experimental import pallas as pl
from jax.experimental.pallas import tpu as pltpu
from jax.experimental.pallas import tpu_sc as plsc

BATCH = 16384
HIST = 50
DIM = 64
LANES = 16
NUM_WORKERS = 32
SAMPLES_PER_TILE = BATCH // NUM_WORKERS
CHUNK = 2
IDX_PER_CHUNK = CHUNK * HIST
NCHUNK = SAMPLES_PER_TILE // CHUNK
INV_HIST = 1.0 / HIST


NBUF = 4


def _sc_body(xs_hbm, table_hbm, out_hbm, idx_v, out_v, *bufs_and_sems):
    rows = bufs_and_sems[:NBUF]
    sems = bufs_and_sems[NBUF:]
    cid = lax.axis_index("c")
    sid = lax.axis_index("s")
    wid = sid * 2 + cid

    pltpu.sync_copy(xs_hbm.at[wid], idx_v)

    def start_gather(j, buf, sem):
        pltpu.async_copy(table_hbm.at[idx_v.at[j]], buf, sem)

    def wait_gather(j, buf, sem):
        pltpu.make_async_copy(table_hbm.at[idx_v.at[j]], buf, sem).wait()

    def reduce_chunk(j, buf):
        for k in range(CHUNK):
            base = k * HIST
            dsls = [pl.ds(d * LANES, LANES) for d in range(DIM // LANES)]
            accs = [buf[base, dsl] for dsl in dsls]
            for r in range(1, HIST):
                for d, dsl in enumerate(dsls):
                    accs[d] = accs[d] + buf[base + r, dsl]
            for d, dsl in enumerate(dsls):
                out_v[j * CHUNK + k, dsl] = accs[d] * INV_HIST

    for b in range(NBUF):
        start_gather(b, rows[b], sems[b])

    @pl.loop(0, NCHUNK, step=NBUF)
    def _(j):
        for b in range(NBUF):
            wait_gather(j + b, rows[b], sems[b])
            reduce_chunk(j + b, rows[b])

            @pl.when(j + b + NBUF < NCHUNK)
            def _(b=b):
                start_gather(j + b + NBUF, rows[b], sems[b])

    base = wid * SAMPLES_PER_TILE
    pltpu.sync_copy(out_v, out_hbm.at[pl.ds(base, SAMPLES_PER_TILE)])


@jax.jit
def kernel(xs, table):
    xs = jnp.reshape(xs.astype(jnp.int32), (NUM_WORKERS, NCHUNK, IDX_PER_CHUNK))
    mesh = plsc.VectorSubcoreMesh(core_axis_name="c", subcore_axis_name="s")
    run = pl.kernel(
        _sc_body,
        out_type=jax.ShapeDtypeStruct((BATCH, DIM), jnp.float32),
        mesh=mesh,
        compiler_params=pltpu.CompilerParams(use_tc_tiling_on_sc=False),
        scratch_types=(
            [
                pltpu.VMEM((NCHUNK, IDX_PER_CHUNK), jnp.int32),
                pltpu.VMEM((SAMPLES_PER_TILE, DIM), jnp.float32),
            ]
            + [pltpu.VMEM((IDX_PER_CHUNK, DIM), jnp.float32)] * NBUF
            + [pltpu.SemaphoreType.DMA] * NBUF
        ),
    )
    return run(xs, table)


# speedup vs baseline: 1.1989x; 1.1989x over previous
import functools

import jax
import jax.numpy as jnp
from jax import lax
from jax.experimental import pallas as pl
from jax.experimental.pallas import tpu as pltpu
from jax.experimental.pallas import tpu_sc as plsc

BATCH = 16384
HIST = 50
DIM = 64
LANES = 16
NUM_WORKERS = 32
SAMPLES_PER_TILE = BATCH // NUM_WORKERS
CHUNK = 2
IDX_PER_CHUNK = CHUNK * HIST
NCHUNK = SAMPLES_PER_TILE // CHUNK
INV_HIST = 1.0 / HIST


NBUF = 4

VOCAB = 1000000
TBLK = 2048
TGRID = -(-VOCAB // TBLK)
MROWS = TGRID * TBLK // 2


def _tc_transpose_body(x_ref, o_ref):
    y = jnp.transpose(x_ref[...], (1, 0))
    o_ref[:, 0:DIM] = y[0 : TBLK // 2, :]
    o_ref[:, DIM : 2 * DIM] = y[TBLK // 2 : TBLK, :]


def _perm(v):
    return ((v >> 11) << 11) + ((v & 1023) << 1) + ((v >> 10) & 1)


def _sc_body(xs_hbm, table_hbm, out_hbm, idx_v, out_v, *bufs_and_sems):
    rows = bufs_and_sems[:NBUF]
    sems = bufs_and_sems[NBUF:]
    cid = lax.axis_index("c")
    sid = lax.axis_index("s")
    wid = sid * 2 + cid

    pltpu.sync_copy(xs_hbm.at[wid], idx_v)

    def start_gather(j, buf, sem):
        pltpu.async_copy(table_hbm.at[idx_v.at[j]], buf, sem)

    def wait_gather(j, buf, sem):
        pltpu.make_async_copy(table_hbm.at[idx_v.at[j]], buf, sem).wait()

    def reduce_chunk(j, buf):
        for k in range(CHUNK):
            base = k * HIST
            dsls = [pl.ds(d * LANES, LANES) for d in range(DIM // LANES)]
            accs = [buf[base, dsl] for dsl in dsls]
            for r in range(1, HIST):
                for d, dsl in enumerate(dsls):
                    accs[d] = accs[d] + buf[base + r, dsl]
            for d, dsl in enumerate(dsls):
                out_v[j * CHUNK + k, dsl] = accs[d] * INV_HIST

    for b in range(NBUF):
        start_gather(b, rows[b], sems[b])

    @pl.loop(0, NCHUNK, step=NBUF)
    def _(j):
        for b in range(NBUF):
            wait_gather(j + b, rows[b], sems[b])
            reduce_chunk(j + b, rows[b])

            @pl.when(j + b + NBUF < NCHUNK)
            def _(b=b):
                start_gather(j + b + NBUF, rows[b], sems[b])

    base = wid * SAMPLES_PER_TILE
    pltpu.sync_copy(out_v, out_hbm.at[pl.ds(base, SAMPLES_PER_TILE)])


@jax.jit
def kernel(xs, table):
    m = pl.pallas_call(
        _tc_transpose_body,
        grid=(TGRID,),
        in_specs=[pl.BlockSpec((DIM, TBLK), lambda i: (0, i))],
        out_specs=pl.BlockSpec((TBLK // 2, 2 * DIM), lambda i: (i, 0)),
        out_shape=jax.ShapeDtypeStruct((MROWS, 2 * DIM), jnp.float32),
    )(table.T)
    m2 = jnp.reshape(m, (2 * MROWS, DIM))
    xs = _perm(xs.astype(jnp.int32))
    xs = jnp.reshape(xs, (NUM_WORKERS, NCHUNK, IDX_PER_CHUNK))
    mesh = plsc.VectorSubcoreMesh(core_axis_name="c", subcore_axis_name="s")
    run = pl.kernel(
        _sc_body,
        out_type=jax.ShapeDtypeStruct((BATCH, DIM), jnp.float32),
        mesh=mesh,
        compiler_params=pltpu.CompilerParams(use_tc_tiling_on_sc=False),
        scratch_types=(
            [
                pltpu.VMEM((NCHUNK, IDX_PER_CHUNK), jnp.int32),
                pltpu.VMEM((SAMPLES_PER_TILE, DIM), jnp.float32),
            ]
            + [pltpu.VMEM((IDX_PER_CHUNK, DIM), jnp.float32)] * NBUF
            + [pltpu.SemaphoreType.DMA] * NBUF
        ),
    )
    return run(xs, m2)


# speedup vs baseline: 1.6592x; 1.3840x over previous
import functools

import jax
import jax.numpy as jnp
from jax import lax
from jax.experimental import pallas as pl
from jax.experimental.pallas import tpu as pltpu
from jax.experimental.pallas import tpu_sc as plsc

BATCH = 16384
HIST = 50
DIM = 64
LANES = 16
NUM_WORKERS = 32
SAMPLES_PER_TILE = BATCH // NUM_WORKERS
CHUNK = 2
IDX_PER_CHUNK = CHUNK * HIST
NCHUNK = SAMPLES_PER_TILE // CHUNK
INV_HIST = 1.0 / HIST


NBUF = 4

VOCAB = 1000000
TBLK = 8192
TGRID = -(-VOCAB // TBLK)
MROWS = TGRID * TBLK // 2


def _tc_transpose_body(x_ref, o_ref):
    y0 = jnp.transpose(x_ref[:, 0 : TBLK // 2], (1, 0))
    y1 = jnp.transpose(x_ref[:, TBLK // 2 : TBLK], (1, 0))
    o_ref[...] = jnp.concatenate([y0, y1], axis=1)


_TS = TBLK.bit_length() - 1


def _perm(v):
    return ((v >> _TS) << _TS) + ((v & (TBLK // 2 - 1)) << 1) + ((v >> (_TS - 1)) & 1)


def _sc_body(xs_hbm, table_hbm, out_hbm, idx_v, out_v, *bufs_and_sems):
    rows = bufs_and_sems[:NBUF]
    sems = bufs_and_sems[NBUF:]
    cid = lax.axis_index("c")
    sid = lax.axis_index("s")
    wid = sid * 2 + cid

    pltpu.sync_copy(xs_hbm.at[wid], idx_v)

    def start_gather(j, buf, sem):
        pltpu.async_copy(table_hbm.at[idx_v.at[j]], buf, sem)

    def wait_gather(j, buf, sem):
        pltpu.make_async_copy(table_hbm.at[idx_v.at[j]], buf, sem).wait()

    def reduce_chunk(j, buf):
        for k in range(CHUNK):
            base = k * HIST
            dsls = [pl.ds(d * LANES, LANES) for d in range(DIM // LANES)]
            accs = [buf[base, dsl] for dsl in dsls]
            for r in range(1, HIST):
                for d, dsl in enumerate(dsls):
                    accs[d] = accs[d] + buf[base + r, dsl]
            for d, dsl in enumerate(dsls):
                out_v[j * CHUNK + k, dsl] = accs[d] * INV_HIST

    for b in range(NBUF):
        start_gather(b, rows[b], sems[b])

    @pl.loop(0, NCHUNK, step=NBUF)
    def _(j):
        for b in range(NBUF):
            wait_gather(j + b, rows[b], sems[b])
            reduce_chunk(j + b, rows[b])

            @pl.when(j + b + NBUF < NCHUNK)
            def _(b=b):
                start_gather(j + b + NBUF, rows[b], sems[b])

    base = wid * SAMPLES_PER_TILE
    pltpu.sync_copy(out_v, out_hbm.at[pl.ds(base, SAMPLES_PER_TILE)])


@jax.jit
def kernel(xs, table):
    m = pl.pallas_call(
        _tc_transpose_body,
        grid=(TGRID,),
        in_specs=[pl.BlockSpec((DIM, TBLK), lambda i: (0, i))],
        out_specs=pl.BlockSpec((TBLK // 2, 2 * DIM), lambda i: (i, 0)),
        out_shape=jax.ShapeDtypeStruct((MROWS, 2 * DIM), jnp.float32),
    )(table.T)
    m2 = jnp.reshape(m, (2 * MROWS, DIM))
    xs = _perm(xs.astype(jnp.int32))
    xs = jnp.reshape(xs, (NUM_WORKERS, NCHUNK, IDX_PER_CHUNK))
    mesh = plsc.VectorSubcoreMesh(core_axis_name="c", subcore_axis_name="s")
    run = pl.kernel(
        _sc_body,
        out_type=jax.ShapeDtypeStruct((BATCH, DIM), jnp.float32),
        mesh=mesh,
        compiler_params=pltpu.CompilerParams(use_tc_tiling_on_sc=False),
        scratch_types=(
            [
                pltpu.VMEM((NCHUNK, IDX_PER_CHUNK), jnp.int32),
                pltpu.VMEM((SAMPLES_PER_TILE, DIM), jnp.float32),
            ]
            + [pltpu.VMEM((IDX_PER_CHUNK, DIM), jnp.float32)] * NBUF
            + [pltpu.SemaphoreType.DMA] * NBUF
        ),
    )
    return run(xs, m2)
